# baseline (speedup 1.0000x reference)
import functools

import jax
import jax.numpy as jnp
from jax import lax
from jax.experimental import pallas as pl
from jax.experimental.pallas import tpu as pltpu
from jax.experimental.pallas import tpu_sc as plsc

B, D, H, W = 4, 64, 64, 64
E = 32
DHW = D * H * W
N = B * DHW
EP = 128

NC, NS = 2, 16
NW = NC * NS
PER_W = N // NW
CHUNK = 2048
N_CHUNKS = PER_W // CHUNK

_mesh = plsc.VectorSubcoreMesh(
    core_axis_name="c", subcore_axis_name="s", num_cores=NC, num_subcores=NS
)


@functools.partial(
    pl.kernel,
    out_type=jax.ShapeDtypeStruct((N, EP), jnp.float32),
    mesh=_mesh,
    scratch_types=[
        pltpu.VMEM((CHUNK,), jnp.int32),
        pltpu.VMEM((CHUNK, E), jnp.float32),
        pltpu.SemaphoreType.DMA,
    ],
    compiler_params=pltpu.CompilerParams(use_tc_tiling_on_sc=False),
)
def _sc_gather(idx_hbm, table_hbm, out_hbm, idx_v, rows_v, sem):
    wid = lax.axis_index("s") * NC + lax.axis_index("c")
    base = wid * PER_W

    def body(i, carry):
        start = base + i * CHUNK
        pltpu.sync_copy(idx_hbm.at[pl.ds(start, CHUNK)], idx_v)
        pltpu.async_copy(table_hbm.at[idx_v], rows_v, sem).wait()
        pltpu.sync_copy(rows_v, out_hbm.at[pl.ds(start, CHUNK), pl.ds(0, E)])
        return carry

    lax.fori_loop(0, N_CHUNKS, body, 0)


_TM = 2048
_TK = DHW // _TM


def _tc_transpose_body(emb_ref, out_ref):
    out_ref[0] = emb_ref[0, :, :E].T


_tc_transpose = pl.pallas_call(
    _tc_transpose_body,
    grid=(B, _TK),
    in_specs=[pl.BlockSpec((1, _TM, EP), lambda b, k: (b, k, 0))],
    out_specs=pl.BlockSpec((1, E, _TM), lambda b, k: (b, 0, k)),
    out_shape=jax.ShapeDtypeStruct((B, E, DHW), jnp.float32),
)


def kernel(v, table):
    idx = v.reshape(N)
    rows = _sc_gather(idx, table)
    out = _tc_transpose(rows.reshape(B, DHW, EP))
    return out.reshape(B, E, D, H, W)


# speedup vs baseline: 1.6387x; 1.6387x over previous
import functools

import jax
import jax.numpy as jnp
from jax import lax
from jax.experimental import pallas as pl
from jax.experimental.pallas import tpu as pltpu
from jax.experimental.pallas import tpu_sc as plsc

B, D, H, W = 4, 64, 64, 64
E = 32
DHW = D * H * W
N = B * DHW

NC, NS = 2, 16
NW = NC * NS
W_PER_B = NW // B
PER_W = DHW // W_PER_B
CHUNK = 1024
N_CHUNKS = PER_W // CHUNK
PITCH = CHUNK + 1

_mesh = plsc.VectorSubcoreMesh(
    core_axis_name="c", subcore_axis_name="s", num_cores=NC, num_subcores=NS
)


@functools.partial(
    pl.kernel,
    out_type=jax.ShapeDtypeStruct((B, E, DHW), jnp.float32),
    mesh=_mesh,
    scratch_types=[
        pltpu.VMEM((CHUNK,), jnp.int32),
        pltpu.VMEM((CHUNK, E), jnp.float32),
        pltpu.VMEM((E, PITCH), jnp.float32),
        pltpu.SemaphoreType.DMA,
    ],
    compiler_params=pltpu.CompilerParams(
        use_tc_tiling_on_sc=False, needs_layout_passes=False
    ),
)
def _sc_fused(idx_hbm, table_hbm, out_hbm, idx_v, rows_v, trans_v, sem):
    wid = lax.axis_index("s") * NC + lax.axis_index("c")
    bb = wid // W_PER_B
    off = (wid % W_PER_B) * PER_W

    e_lo = lax.iota(jnp.int32, 16)
    e_hi = e_lo + 16

    def body(i, carry):
        p0 = off + i * CHUNK
        pltpu.sync_copy(idx_hbm.at[pl.ds(bb * DHW + p0, CHUNK)], idx_v)
        pltpu.async_copy(table_hbm.at[idx_v], rows_v, sem).wait()

        @pl.loop(0, CHUNK, unroll=8)
        def _transpose(j):
            jv = jnp.full((16,), j, jnp.int32)
            r0 = rows_v[j, pl.ds(0, 16)]
            r1 = rows_v[j, pl.ds(16, 16)]
            plsc.store_scatter(trans_v, [e_lo, jv], r0)
            plsc.store_scatter(trans_v, [e_hi, jv], r1)

        pltpu.sync_copy(
            trans_v.at[:, pl.ds(0, CHUNK)],
            out_hbm.at[bb, :, pl.ds(p0, CHUNK)],
        )
        return carry

    lax.fori_loop(0, N_CHUNKS, body, 0)


def kernel(v, table):
    idx = v.reshape(N)
    out = _sc_fused(idx, table)
    return out.reshape(B, E, D, H, W)


# speedup vs baseline: 1.8577x; 1.1336x over previous
import functools

import jax
import jax.numpy as jnp
from jax import lax
from jax.experimental import pallas as pl
from jax.experimental.pallas import tpu as pltpu
from jax.experimental.pallas import tpu_sc as plsc

B, D, H, W = 4, 64, 64, 64
E = 32
DHW = D * H * W
N = B * DHW

NC, NS = 2, 16
NW = NC * NS
W_PER_B = NW // B
PER_W = DHW // W_PER_B
CHUNK = 1024
N_CHUNKS = PER_W // CHUNK
PITCH = CHUNK + 1

_mesh = plsc.VectorSubcoreMesh(
    core_axis_name="c", subcore_axis_name="s", num_cores=NC, num_subcores=NS
)


@functools.partial(
    pl.kernel,
    out_type=jax.ShapeDtypeStruct((B, E, DHW), jnp.float32),
    mesh=_mesh,
    scratch_types=[
        pltpu.VMEM((CHUNK,), jnp.int32),
        pltpu.VMEM((CHUNK,), jnp.int32),
        pltpu.VMEM((CHUNK, E), jnp.float32),
        pltpu.VMEM((CHUNK, E), jnp.float32),
        pltpu.VMEM((E, PITCH), jnp.float32),
        pltpu.SemaphoreType.DMA,
        pltpu.SemaphoreType.DMA,
    ],
    compiler_params=pltpu.CompilerParams(
        use_tc_tiling_on_sc=False, needs_layout_passes=False
    ),
)
def _sc_fused(idx_hbm, table_hbm, out_hbm, idx_v0, idx_v1, rows_v0, rows_v1,
              trans_v, sem0, sem1):
    wid = lax.axis_index("s") * NC + lax.axis_index("c")
    bb = wid // W_PER_B
    off = (wid % W_PER_B) * PER_W

    e_lo = lax.iota(jnp.int32, 16)
    e_hi = e_lo + 16

    def start_gather(k, idx_v, rows_v, sem):
        pltpu.sync_copy(idx_hbm.at[pl.ds(bb * DHW + off + k * CHUNK, CHUNK)],
                        idx_v)
        pltpu.async_copy(table_hbm.at[idx_v], rows_v, sem)

    def finish_chunk(k, idx_v, rows_v, sem):
        pltpu.make_async_copy(table_hbm.at[idx_v], rows_v, sem).wait()

        @pl.loop(0, CHUNK, unroll=8)
        def _transpose(j):
            jv = jnp.full((16,), j, jnp.int32)
            r0 = rows_v[j, pl.ds(0, 16)]
            r1 = rows_v[j, pl.ds(16, 16)]
            plsc.store_scatter(trans_v, [e_lo, jv], r0)
            plsc.store_scatter(trans_v, [e_hi, jv], r1)

        pltpu.sync_copy(
            trans_v.at[:, pl.ds(0, CHUNK)],
            out_hbm.at[bb, :, pl.ds(off + k * CHUNK, CHUNK)],
        )

    start_gather(0, idx_v0, rows_v0, sem0)

    @pl.loop(0, N_CHUNKS, step=2)
    def _pipeline(i):
        start_gather(i + 1, idx_v1, rows_v1, sem1)
        finish_chunk(i, idx_v0, rows_v0, sem0)

        @pl.when(i + 2 < N_CHUNKS)
        def _():
            start_gather(i + 2, idx_v0, rows_v0, sem0)

        finish_chunk(i + 1, idx_v1, rows_v1, sem1)


def kernel(v, table):
    idx = v.reshape(N)
    out = _sc_fused(idx, table)
    return out.reshape(B, E, D, H, W)


# speedup vs baseline: 2.1078x; 1.1346x over previous
import functools

import jax
import jax.numpy as jnp
from jax import lax
from jax.experimental import pallas as pl
from jax.experimental.pallas import tpu as pltpu
from jax.experimental.pallas import tpu_sc as plsc

B, D, H, W = 4, 64, 64, 64
E = 32
DHW = D * H * W
N = B * DHW

NC, NS = 2, 16
NW = NC * NS
W_PER_B = NW // B
PER_W = DHW // W_PER_B
CHUNK = 512
N_CHUNKS = PER_W // CHUNK
PITCH = CHUNK + 1

_mesh = plsc.VectorSubcoreMesh(
    core_axis_name="c", subcore_axis_name="s", num_cores=NC, num_subcores=NS
)


@functools.partial(
    pl.kernel,
    out_type=jax.ShapeDtypeStruct((B, E, DHW), jnp.float32),
    mesh=_mesh,
    scratch_types=[
        pltpu.VMEM((PER_W,), jnp.int32),
        pltpu.VMEM((CHUNK, E), jnp.float32),
        pltpu.VMEM((CHUNK, E), jnp.float32),
        pltpu.VMEM((E, PITCH), jnp.float32),
        pltpu.VMEM((E, PITCH), jnp.float32),
        pltpu.SemaphoreType.DMA,
        pltpu.SemaphoreType.DMA,
        pltpu.SemaphoreType.DMA,
        pltpu.SemaphoreType.DMA,
    ],
    compiler_params=pltpu.CompilerParams(
        use_tc_tiling_on_sc=False, needs_layout_passes=False
    ),
)
def _sc_fused(idx_hbm, table_hbm, out_hbm, idx_all, rows_v0, rows_v1,
              trans_v0, trans_v1, sem0, sem1, osem0, osem1):
    wid = lax.axis_index("s") * NC + lax.axis_index("c")
    bb = wid // W_PER_B
    off = (wid % W_PER_B) * PER_W

    e_lo = lax.iota(jnp.int32, 16)
    e_hi = e_lo + 16

    pltpu.sync_copy(idx_hbm.at[pl.ds(bb * DHW + off, PER_W)], idx_all)

    def start_gather(k, rows_v, sem):
        pltpu.async_copy(
            table_hbm.at[idx_all.at[pl.ds(k * CHUNK, CHUNK)]], rows_v, sem)

    def finish_chunk(k, rows_v, sem, trans_v, osem, wait_osem):
        pltpu.make_async_copy(
            table_hbm.at[idx_all.at[pl.ds(0, CHUNK)]], rows_v, sem).wait()

        if wait_osem is not None:
            @pl.when(wait_osem)
            def _():
                pltpu.make_async_copy(
                    trans_v.at[:, pl.ds(0, CHUNK)],
                    out_hbm.at[bb, :, pl.ds(off, CHUNK)], osem).wait()

        @pl.loop(0, CHUNK, unroll=8)
        def _transpose(j):
            jv = jnp.full((16,), j, jnp.int32)
            r0 = rows_v[j, pl.ds(0, 16)]
            r1 = rows_v[j, pl.ds(16, 16)]
            plsc.store_scatter(trans_v, [e_lo, jv], r0)
            plsc.store_scatter(trans_v, [e_hi, jv], r1)

        pltpu.async_copy(
            trans_v.at[:, pl.ds(0, CHUNK)],
            out_hbm.at[bb, :, pl.ds(off + k * CHUNK, CHUNK)], osem)

    start_gather(0, rows_v0, sem0)

    @pl.loop(0, N_CHUNKS, step=2)
    def _pipeline(i):
        start_gather(i + 1, rows_v1, sem1)
        finish_chunk(i, rows_v0, sem0, trans_v0, osem0, i >= 2)

        @pl.when(i + 2 < N_CHUNKS)
        def _():
            start_gather(i + 2, rows_v0, sem0)

        finish_chunk(i + 1, rows_v1, sem1, trans_v1, osem1, i >= 2)

    for trans_v, osem in ((trans_v0, osem0), (trans_v1, osem1)):
        pltpu.make_async_copy(
            trans_v.at[:, pl.ds(0, CHUNK)],
            out_hbm.at[bb, :, pl.ds(off, CHUNK)], osem).wait()


def kernel(v, table):
    idx = v.reshape(N)
    out = _sc_fused(idx, table)
    return out.reshape(B, E, D, H, W)


# speedup vs baseline: 2.4121x; 1.1444x over previous
import functools

import jax
import jax.numpy as jnp
from jax import lax
from jax.experimental import pallas as pl
from jax.experimental.pallas import tpu as pltpu
from jax.experimental.pallas import tpu_sc as plsc

B, D, H, W = 4, 64, 64, 64
E = 32
DHW = D * H * W
N = B * DHW

NC, NS = 2, 16
NW = NC * NS
W_PER_B = NW // B
PER_W = DHW // W_PER_B
CHUNK = 512
N_CHUNKS = PER_W // CHUNK
PITCH = CHUNK + 1

_mesh = plsc.VectorSubcoreMesh(
    core_axis_name="c", subcore_axis_name="s", num_cores=NC, num_subcores=NS
)


@functools.partial(
    pl.kernel,
    out_type=jax.ShapeDtypeStruct((B, E, DHW), jnp.float32),
    mesh=_mesh,
    scratch_types=[
        pltpu.VMEM((PER_W,), jnp.int32),
        pltpu.VMEM((CHUNK, E), jnp.float32),
        pltpu.VMEM((CHUNK, E), jnp.float32),
        pltpu.VMEM((E, PITCH), jnp.float32),
        pltpu.VMEM((E, PITCH), jnp.float32),
        pltpu.SemaphoreType.DMA,
        pltpu.SemaphoreType.DMA,
        pltpu.SemaphoreType.DMA,
        pltpu.SemaphoreType.DMA,
    ],
    compiler_params=pltpu.CompilerParams(
        use_tc_tiling_on_sc=False, needs_layout_passes=False
    ),
)
def _sc_fused(idx_hbm, table_hbm, out_hbm, idx_all, rows_v0, rows_v1,
              trans_v0, trans_v1, sem0, sem1, osem0, osem1):
    wid = lax.axis_index("s") * NC + lax.axis_index("c")
    bb = wid // W_PER_B
    off = (wid % W_PER_B) * PER_W

    e_lo = lax.iota(jnp.int32, 16)
    e_hi = e_lo + 16

    pltpu.sync_copy(idx_hbm.at[pl.ds(bb * DHW + off, PER_W)], idx_all)

    def start_gather(k, rows_v, sem):
        pltpu.async_copy(
            table_hbm.at[idx_all.at[pl.ds(k * CHUNK, CHUNK)]], rows_v, sem)

    def finish_chunk(k, rows_v, sem, trans_v, osem, wait_osem):
        pltpu.make_async_copy(
            table_hbm.at[idx_all.at[pl.ds(0, CHUNK)]], rows_v, sem).wait()

        if wait_osem is not None:
            @pl.when(wait_osem)
            def _():
                pltpu.make_async_copy(
                    trans_v.at[:, pl.ds(0, CHUNK)],
                    out_hbm.at[bb, :, pl.ds(off, CHUNK)], osem).wait()

        @functools.partial(plsc.parallel_loop, 0, CHUNK, unroll=16)
        def _transpose(j):
            jv = jnp.full((16,), j, jnp.int32)
            r0 = rows_v[j, pl.ds(0, 16)]
            r1 = rows_v[j, pl.ds(16, 16)]
            plsc.store_scatter(trans_v, [e_lo, jv], r0)
            plsc.store_scatter(trans_v, [e_hi, jv], r1)

        pltpu.async_copy(
            trans_v.at[:, pl.ds(0, CHUNK)],
            out_hbm.at[bb, :, pl.ds(off + k * CHUNK, CHUNK)], osem)

    start_gather(0, rows_v0, sem0)

    @pl.loop(0, N_CHUNKS, step=2)
    def _pipeline(i):
        start_gather(i + 1, rows_v1, sem1)
        finish_chunk(i, rows_v0, sem0, trans_v0, osem0, i >= 2)

        @pl.when(i + 2 < N_CHUNKS)
        def _():
            start_gather(i + 2, rows_v0, sem0)

        finish_chunk(i + 1, rows_v1, sem1, trans_v1, osem1, i >= 2)

    for trans_v, osem in ((trans_v0, osem0), (trans_v1, osem1)):
        pltpu.make_async_copy(
            trans_v.at[:, pl.ds(0, CHUNK)],
            out_hbm.at[bb, :, pl.ds(off, CHUNK)], osem).wait()


def kernel(v, table):
    idx = v.reshape(N)
    out = _sc_fused(idx, table)
    return out.reshape(B, E, D, H, W)
